# baseline (speedup 1.0000x reference)
import functools

import jax
import jax.numpy as jnp
from jax import lax
from jax.experimental import pallas as pl
from jax.experimental.pallas import tpu as pltpu
from jax.experimental.pallas import tpu_sc as plsc

NC = 2
NS = 16
LANES = 16

CHUNK = 128


def _sc_mesh():
    return plsc.VectorSubcoreMesh(
        core_axis_name="c", subcore_axis_name="s", num_cores=NC, num_subcores=NS
    )


def _degree_kernel(n_nodes, n_chunks):
    acc_len = n_nodes + CHUNK

    @functools.partial(
        pl.kernel,
        out_type=jax.ShapeDtypeStruct((NC * n_nodes,), jnp.float32),
        mesh=_sc_mesh(),
        scratch_types=[
            pltpu.VMEM((n_chunks, CHUNK), jnp.int32),
            pltpu.VMEM((CHUNK,), jnp.float32),
            pltpu.VMEM((1024,), jnp.float32),
            pltpu.VMEM_SHARED((acc_len,), jnp.float32),
            pltpu.SemaphoreType.DMA,
        ],
    )
    def deg_kernel(dst3_hbm, out_hbm, didx, ones_v, zbuf, acc_sh, sem):
        c = lax.axis_index("c")
        s = lax.axis_index("s")
        tile = c * NS + s

        pltpu.sync_copy(dst3_hbm.at[tile], didx)

        def fill_ones(i, _):
            ones_v[pl.ds(i * LANES, LANES)] = jnp.ones((LANES,), jnp.float32)
            return 0
        lax.fori_loop(0, CHUNK // LANES, fill_ones, 0)

        def fill_zero(i, _):
            zbuf[pl.ds(i * LANES, LANES)] = jnp.zeros((LANES,), jnp.float32)
            return 0
        lax.fori_loop(0, 1024 // LANES, fill_zero, 0)

        @pl.when(s == 0)
        def _():
            n_z = n_nodes // 1024

            def zero_acc(i, _):
                pltpu.sync_copy(zbuf, acc_sh.at[pl.ds(i * 1024, 1024)])
                return 0
            lax.fori_loop(0, n_z, zero_acc, 0)
            rem = n_nodes - n_z * 1024
            if rem:
                pltpu.sync_copy(
                    zbuf.at[pl.ds(0, rem)], acc_sh.at[pl.ds(n_z * 1024, rem)]
                )

        plsc.subcore_barrier()

        def fire(j, _):
            pltpu.async_copy(ones_v, acc_sh.at[didx.at[j]], sem, add=True)
            return 0
        lax.fori_loop(0, n_chunks, fire, 0)

        def drain(j, _):
            pltpu.make_async_copy(ones_v, acc_sh.at[didx.at[0]], sem).wait()
            return 0
        lax.fori_loop(0, n_chunks, drain, 0)

        plsc.subcore_barrier()

        n_oc = n_nodes // 1024
        oc_per_tile = (n_oc + NS - 1) // NS

        def copy_out(i, _):
            k = s + i * NS

            @pl.when(k < n_oc)
            def _():
                pltpu.sync_copy(acc_sh.at[pl.ds(k * 1024, 1024)], zbuf)
                pltpu.sync_copy(
                    zbuf, out_hbm.at[pl.ds(c * n_nodes + k * 1024, 1024)]
                )
            return 0
        lax.fori_loop(0, oc_per_tile, copy_out, 0)
        rem = n_nodes - n_oc * 1024
        if rem:
            @pl.when(s == NS - 1)
            def _():
                pltpu.sync_copy(
                    acc_sh.at[pl.ds(n_oc * 1024, rem)], zbuf.at[pl.ds(0, rem)]
                )
                pltpu.sync_copy(
                    zbuf.at[pl.ds(0, rem)],
                    out_hbm.at[pl.ds(c * n_nodes + n_oc * 1024, rem)],
                )

    return deg_kernel


def _scatter_kernel(n_nodes, n_chunks, d):
    assert n_chunks % 2 == 0
    n_iters = n_chunks // 2
    acc_rows = n_nodes + CHUNK
    nz_full = acc_rows // CHUNK
    nz_tail = acc_rows - nz_full * CHUNK
    n_row_chunks = n_nodes // CHUNK
    row_tail = n_nodes - n_row_chunks * CHUNK
    chunks_per_tile = (nz_full + NS - 1) // NS

    @functools.partial(
        pl.kernel,
        out_type=jax.ShapeDtypeStruct((NC * n_nodes, d), jnp.float32),
        mesh=_sc_mesh(),
        scratch_types=[
            pltpu.VMEM((CHUNK,), jnp.int32),
            pltpu.VMEM((CHUNK,), jnp.int32),
            pltpu.VMEM((CHUNK,), jnp.int32),
            pltpu.VMEM((CHUNK,), jnp.int32),
            pltpu.VMEM((CHUNK, d), jnp.float32),
            pltpu.VMEM((CHUNK, d), jnp.float32),
            pltpu.VMEM_SHARED((acc_rows, d), jnp.float32),
            pltpu.SemaphoreType.DMA,
        ],
    )
    def scat_kernel(src_hbm, dst_hbm, g_hbm, out_hbm,
                    sidx0, sidx1, didx0, didx1, rows0, rows1, acc_sh,
                    gsem):
        c = lax.axis_index("c")
        s = lax.axis_index("s")
        tile = c * NS + s
        base = tile * n_chunks * CHUNK

        def fill_row(i, _):
            def fill_lane(j, _):
                rows0[i, pl.ds(j * LANES, LANES)] = jnp.zeros((LANES,), jnp.float32)
                return 0
            lax.fori_loop(0, d // LANES, fill_lane, 0)
            return 0
        lax.fori_loop(0, CHUNK, fill_row, 0)

        def zero_rows(i, _):
            k = s + i * NS

            @pl.when(k < nz_full)
            def _():
                pltpu.sync_copy(rows0, acc_sh.at[pl.ds(k * CHUNK, CHUNK)])
            return 0
        lax.fori_loop(0, chunks_per_tile, zero_rows, 0)
        if nz_tail:
            @pl.when(s == NS - 1)
            def _():
                pltpu.sync_copy(
                    rows0.at[pl.ds(0, nz_tail)],
                    acc_sh.at[pl.ds(nz_full * CHUNK, nz_tail)],
                )

        plsc.subcore_barrier()

        def gather(slot, buf):
            pltpu.async_copy(g_hbm.at[slot], buf, gsem)

        def wait_gather(slot, buf):
            pltpu.make_async_copy(g_hbm.at[pl.ds(0, CHUNK)], buf, gsem).wait()

        def body(j, _):
            off = base + j * CHUNK
            pltpu.sync_copy(src_hbm.at[pl.ds(off, CHUNK)], sidx0)
            pltpu.sync_copy(dst_hbm.at[pl.ds(off, CHUNK)], didx0)
            pltpu.async_copy(g_hbm.at[sidx0], rows0, gsem).wait()
            pltpu.sync_copy(rows0, acc_sh.at[didx0], add=True)
            return 0
        lax.fori_loop(0, n_chunks, body, 0)

        plsc.subcore_barrier()

        out_base = c * n_nodes

        def copy_out(i, _):
            k = s + i * NS

            @pl.when(k < n_row_chunks)
            def _():
                pltpu.sync_copy(
                    acc_sh.at[pl.ds(k * CHUNK, CHUNK)],
                    out_hbm.at[pl.ds(out_base + k * CHUNK, CHUNK)],
                )
            return 0
        lax.fori_loop(0, chunks_per_tile, copy_out, 0)
        if row_tail:
            @pl.when(s == 0)
            def _():
                pltpu.sync_copy(
                    acc_sh.at[pl.ds(n_row_chunks * CHUNK, row_tail)],
                    out_hbm.at[pl.ds(out_base + n_row_chunks * CHUNK, row_tail)],
                )

    return scat_kernel


def _gW_body(x_ref, w_ref, degp_ref, g_ref):
    n = x_ref.shape[0]
    h = jnp.dot(x_ref[...], w_ref[...], preferred_element_type=jnp.float32)
    deg = degp_ref[0:n] + degp_ref[n:2 * n] + 1.0
    dis = lax.rsqrt(deg)
    g_ref[...] = h * dis[:, None]


def _epilogue_body(s_ref, g_ref, degp_ref, b_ref, gamma_ref, beta_ref, y_ref):
    n = g_ref.shape[0]
    deg = degp_ref[0:n] + degp_ref[n:2 * n] + 1.0
    dis = lax.rsqrt(deg)
    total = s_ref[0:n, :] + s_ref[n:2 * n, :] + g_ref[...]
    out = total * dis[:, None] + b_ref[...][None, :]
    mean = jnp.mean(out, axis=0)
    var = jnp.mean((out - mean[None, :]) ** 2, axis=0)
    y = gamma_ref[...][None, :] * (out - mean[None, :]) * lax.rsqrt(
        var[None, :] + 1e-5
    ) + beta_ref[...][None, :]
    y_ref[...] = jnp.maximum(y, 0.0)


def kernel(x, edge_index, W, b, gamma, beta):
    n_nodes, d_in = x.shape
    d_out = W.shape[1]
    n_edges = edge_index.shape[1]
    src = edge_index[0]
    dst = edge_index[1]

    n_tiles = NC * NS
    if n_edges % n_tiles:
        flat_pad = n_tiles - n_edges % n_tiles
        src = jnp.concatenate([src, jnp.zeros((flat_pad,), jnp.int32)])
        dst = jnp.concatenate([dst, jnp.full((flat_pad,), n_nodes, jnp.int32)])
        n_edges += flat_pad
    per_tile = n_edges // n_tiles
    n_chunks = -(-per_tile // CHUNK)
    n_chunks += n_chunks % 2
    pad_pt = n_chunks * CHUNK - per_tile
    src2 = src.reshape(n_tiles, per_tile)
    dst2 = dst.reshape(n_tiles, per_tile)
    dummy = n_nodes + jnp.arange(pad_pt, dtype=jnp.int32) % CHUNK
    src3 = jnp.concatenate(
        [src2, jnp.zeros((n_tiles, pad_pt), jnp.int32)], axis=1
    ).reshape(n_tiles, n_chunks, CHUNK)
    dst3 = jnp.concatenate(
        [dst2, jnp.broadcast_to(dummy[None, :], (n_tiles, pad_pt))], axis=1
    ).reshape(n_tiles, n_chunks, CHUNK)

    degp = _degree_kernel(n_nodes, n_chunks)(dst3)

    g = pl.pallas_call(
        _gW_body,
        out_shape=jax.ShapeDtypeStruct((n_nodes, d_out), jnp.float32),
    )(x, W, degp)

    s_partial = _scatter_kernel(n_nodes, n_chunks, d_out)(
        src3.reshape(-1), dst3.reshape(-1), g
    )

    y = pl.pallas_call(
        _epilogue_body,
        out_shape=jax.ShapeDtypeStruct((n_nodes, d_out), jnp.float32),
    )(s_partial, g, degp, b, gamma, beta)
    return y


# speedup vs baseline: 1.7106x; 1.7106x over previous
import functools

import jax
import jax.numpy as jnp
from jax import lax
from jax.experimental import pallas as pl
from jax.experimental.pallas import tpu as pltpu
from jax.experimental.pallas import tpu_sc as plsc

NC = 2
NS = 16
LANES = 16

CHUNK = 128


def _sc_mesh():
    return plsc.VectorSubcoreMesh(
        core_axis_name="c", subcore_axis_name="s", num_cores=NC, num_subcores=NS
    )


def _degree_kernel(n_nodes, n_edges):
    edges_per_tile = n_edges // (NC * NS)
    n_full = edges_per_tile // CHUNK
    tail = edges_per_tile - n_full * CHUNK

    @functools.partial(
        pl.kernel,
        out_type=jax.ShapeDtypeStruct((NC * n_nodes,), jnp.float32),
        mesh=_sc_mesh(),
        scratch_types=[
            pltpu.VMEM((CHUNK,), jnp.int32),
            pltpu.VMEM((CHUNK,), jnp.float32),
            pltpu.VMEM((16,), jnp.int32),
            pltpu.VMEM((1024,), jnp.float32),
            pltpu.VMEM_SHARED((n_nodes,), jnp.float32),
        ],
    )
    def deg_kernel(dst_hbm, out_hbm, idx_v, ones_v, idx_t, zbuf, acc_sh):
        c = lax.axis_index("c")
        s = lax.axis_index("s")
        tile = c * NS + s
        base = tile * edges_per_tile

        def fill_ones(i, _):
            ones_v[pl.ds(i * LANES, LANES)] = jnp.ones((LANES,), jnp.float32)
            return 0
        lax.fori_loop(0, CHUNK // LANES, fill_ones, 0)

        def fill_zero(i, _):
            zbuf[pl.ds(i * LANES, LANES)] = jnp.zeros((LANES,), jnp.float32)
            return 0
        lax.fori_loop(0, 1024 // LANES, fill_zero, 0)

        @pl.when(s == 0)
        def _():
            n_z = n_nodes // 1024

            def zero_acc(i, _):
                pltpu.sync_copy(zbuf, acc_sh.at[pl.ds(i * 1024, 1024)])
                return 0
            lax.fori_loop(0, n_z, zero_acc, 0)
            rem = n_nodes - n_z * 1024
            if rem:
                pltpu.sync_copy(
                    zbuf.at[pl.ds(0, rem)], acc_sh.at[pl.ds(n_z * 1024, rem)]
                )

        plsc.subcore_barrier()

        def body(j, _):
            off = base + j * CHUNK
            pltpu.sync_copy(dst_hbm.at[pl.ds(off, CHUNK)], idx_v)
            pltpu.sync_copy(ones_v, acc_sh.at[idx_v], add=True)
            return 0
        lax.fori_loop(0, n_full, body, 0)

        if tail:
            pltpu.sync_copy(
                dst_hbm.at[pl.ds(base + n_full * CHUNK, tail)], idx_t
            )
            pltpu.sync_copy(ones_v.at[pl.ds(0, tail)], acc_sh.at[idx_t], add=True)

        plsc.subcore_barrier()

        n_oc = n_nodes // 1024
        oc_per_tile = (n_oc + NS - 1) // NS

        def copy_out(i, _):
            k = s + i * NS

            @pl.when(k < n_oc)
            def _():
                pltpu.sync_copy(acc_sh.at[pl.ds(k * 1024, 1024)], zbuf)
                pltpu.sync_copy(
                    zbuf, out_hbm.at[pl.ds(c * n_nodes + k * 1024, 1024)]
                )
            return 0
        lax.fori_loop(0, oc_per_tile, copy_out, 0)
        rem = n_nodes - n_oc * 1024
        if rem:
            @pl.when(s == NS - 1)
            def _():
                pltpu.sync_copy(acc_sh.at[pl.ds(n_oc * 1024, rem)], zbuf.at[pl.ds(0, rem)])
                pltpu.sync_copy(
                    zbuf.at[pl.ds(0, rem)],
                    out_hbm.at[pl.ds(c * n_nodes + n_oc * 1024, rem)],
                )

    return deg_kernel


def _scatter_kernel(n_nodes, n_edges, d):
    edges_per_tile = n_edges // (NC * NS)
    n_full = edges_per_tile // CHUNK
    tail = edges_per_tile - n_full * CHUNK
    n_row_chunks = n_nodes // CHUNK
    row_tail = n_nodes - n_row_chunks * CHUNK
    chunks_per_tile = (n_row_chunks + NS - 1) // NS

    @functools.partial(
        pl.kernel,
        out_type=jax.ShapeDtypeStruct((NC * n_nodes, d), jnp.float32),
        mesh=_sc_mesh(),
        scratch_types=[
            pltpu.VMEM((CHUNK,), jnp.int32),
            pltpu.VMEM((CHUNK,), jnp.int32),
            pltpu.VMEM((CHUNK, d), jnp.float32),
            pltpu.VMEM((16,), jnp.int32),
            pltpu.VMEM((16,), jnp.int32),
            pltpu.VMEM_SHARED((n_nodes, d), jnp.float32),
            pltpu.SemaphoreType.DMA,
        ],
    )
    def scat_kernel(src_hbm, dst_hbm, g_hbm, out_hbm,
                    sidx, didx, rows, sidx_t, didx_t, acc_sh, sem):
        c = lax.axis_index("c")
        s = lax.axis_index("s")
        tile = c * NS + s
        base = tile * edges_per_tile

        def fill_row(i, _):
            def fill_lane(j, _):
                rows[i, pl.ds(j * LANES, LANES)] = jnp.zeros((LANES,), jnp.float32)
                return 0
            lax.fori_loop(0, d // LANES, fill_lane, 0)
            return 0
        lax.fori_loop(0, CHUNK, fill_row, 0)

        def zero_rows(i, _):
            k = s + i * NS

            @pl.when(k < n_row_chunks)
            def _():
                pltpu.sync_copy(rows, acc_sh.at[pl.ds(k * CHUNK, CHUNK)])
            return 0
        lax.fori_loop(0, chunks_per_tile, zero_rows, 0)
        if row_tail:
            @pl.when(s == 0)
            def _():
                pltpu.sync_copy(
                    rows.at[pl.ds(0, row_tail)],
                    acc_sh.at[pl.ds(n_row_chunks * CHUNK, row_tail)],
                )

        plsc.subcore_barrier()

        def body(j, _):
            off = base + j * CHUNK
            pltpu.sync_copy(src_hbm.at[pl.ds(off, CHUNK)], sidx)
            pltpu.sync_copy(dst_hbm.at[pl.ds(off, CHUNK)], didx)
            pltpu.async_copy(g_hbm.at[sidx], rows, sem).wait()
            pltpu.sync_copy(rows, acc_sh.at[didx], add=True)
            return 0
        lax.fori_loop(0, n_full, body, 0)

        if tail:
            off = base + n_full * CHUNK
            pltpu.sync_copy(src_hbm.at[pl.ds(off, tail)], sidx_t)
            pltpu.sync_copy(dst_hbm.at[pl.ds(off, tail)], didx_t)
            pltpu.async_copy(
                g_hbm.at[sidx_t], rows.at[pl.ds(0, tail)], sem
            ).wait()
            pltpu.sync_copy(rows.at[pl.ds(0, tail)], acc_sh.at[didx_t], add=True)

        plsc.subcore_barrier()

        out_base = c * n_nodes

        def copy_out(i, _):
            k = s + i * NS

            @pl.when(k < n_row_chunks)
            def _():
                pltpu.sync_copy(
                    acc_sh.at[pl.ds(k * CHUNK, CHUNK)],
                    out_hbm.at[pl.ds(out_base + k * CHUNK, CHUNK)],
                )
            return 0
        lax.fori_loop(0, chunks_per_tile, copy_out, 0)
        if row_tail:
            @pl.when(s == 0)
            def _():
                pltpu.sync_copy(
                    acc_sh.at[pl.ds(n_row_chunks * CHUNK, row_tail)],
                    out_hbm.at[pl.ds(out_base + n_row_chunks * CHUNK, row_tail)],
                )

    return scat_kernel


def _gW_body(x_ref, w_ref, degp_ref, g_ref):
    n = x_ref.shape[0]
    h = jnp.dot(x_ref[...], w_ref[...], preferred_element_type=jnp.float32)
    deg = degp_ref[0:n] + degp_ref[n:2 * n] + 1.0
    dis = lax.rsqrt(deg)
    g_ref[...] = h * dis[:, None]


def _epilogue_body(s_ref, g_ref, degp_ref, b_ref, gamma_ref, beta_ref, y_ref):
    n = g_ref.shape[0]
    deg = degp_ref[0:n] + degp_ref[n:2 * n] + 1.0
    dis = lax.rsqrt(deg)
    total = s_ref[0:n, :] + s_ref[n:2 * n, :] + g_ref[...]
    out = total * dis[:, None] + b_ref[...][None, :]
    mean = jnp.mean(out, axis=0)
    var = jnp.mean((out - mean[None, :]) ** 2, axis=0)
    y = gamma_ref[...][None, :] * (out - mean[None, :]) * lax.rsqrt(
        var[None, :] + 1e-5
    ) + beta_ref[...][None, :]
    y_ref[...] = jnp.maximum(y, 0.0)


def kernel(x, edge_index, W, b, gamma, beta):
    n_nodes, d_in = x.shape
    d_out = W.shape[1]
    n_edges = edge_index.shape[1]
    src = edge_index[0]
    dst = edge_index[1]

    degp = _degree_kernel(n_nodes, n_edges)(dst)

    g = pl.pallas_call(
        _gW_body,
        out_shape=jax.ShapeDtypeStruct((n_nodes, d_out), jnp.float32),
    )(x, W, degp)

    s_partial = _scatter_kernel(n_nodes, n_edges, d_out)(src, dst, g)

    y = pl.pallas_call(
        _epilogue_body,
        out_shape=jax.ShapeDtypeStruct((n_nodes, d_out), jnp.float32),
    )(s_partial, g, degp, b, gamma, beta)
    return y


# speedup vs baseline: 1.9227x; 1.1240x over previous
import functools

import jax
import jax.numpy as jnp
from jax import lax
from jax.experimental import pallas as pl
from jax.experimental.pallas import tpu as pltpu
from jax.experimental.pallas import tpu_sc as plsc

NC = 2
NS = 16
LANES = 16

CHUNK = 128


def _sc_mesh():
    return plsc.VectorSubcoreMesh(
        core_axis_name="c", subcore_axis_name="s", num_cores=NC, num_subcores=NS
    )


def _degree_kernel(n_nodes, n_chunks):
    acc_len = n_nodes + CHUNK

    @functools.partial(
        pl.kernel,
        out_type=jax.ShapeDtypeStruct((NC * n_nodes,), jnp.float32),
        mesh=_sc_mesh(),
        scratch_types=[
            pltpu.VMEM((n_chunks, CHUNK), jnp.int32),
            pltpu.VMEM((CHUNK,), jnp.float32),
            pltpu.VMEM((1024,), jnp.float32),
            pltpu.VMEM_SHARED((acc_len,), jnp.float32),
            pltpu.SemaphoreType.DMA,
        ],
    )
    def deg_kernel(dst3_hbm, out_hbm, didx, ones_v, zbuf, acc_sh, sem):
        c = lax.axis_index("c")
        s = lax.axis_index("s")
        tile = c * NS + s

        pltpu.sync_copy(dst3_hbm.at[tile], didx)

        def fill_ones(i, _):
            ones_v[pl.ds(i * LANES, LANES)] = jnp.ones((LANES,), jnp.float32)
            return 0
        lax.fori_loop(0, CHUNK // LANES, fill_ones, 0)

        def fill_zero(i, _):
            zbuf[pl.ds(i * LANES, LANES)] = jnp.zeros((LANES,), jnp.float32)
            return 0
        lax.fori_loop(0, 1024 // LANES, fill_zero, 0)

        @pl.when(s == 0)
        def _():
            n_z = n_nodes // 1024

            def zero_acc(i, _):
                pltpu.sync_copy(zbuf, acc_sh.at[pl.ds(i * 1024, 1024)])
                return 0
            lax.fori_loop(0, n_z, zero_acc, 0)
            rem = n_nodes - n_z * 1024
            if rem:
                pltpu.sync_copy(
                    zbuf.at[pl.ds(0, rem)], acc_sh.at[pl.ds(n_z * 1024, rem)]
                )

        plsc.subcore_barrier()

        def fire(j, _):
            pltpu.async_copy(ones_v, acc_sh.at[didx.at[j]], sem, add=True)
            return 0
        lax.fori_loop(0, n_chunks, fire, 0)

        def drain(j, _):
            pltpu.make_async_copy(ones_v, acc_sh.at[didx.at[0]], sem).wait()
            return 0
        lax.fori_loop(0, n_chunks, drain, 0)

        plsc.subcore_barrier()

        n_oc = n_nodes // 1024
        oc_per_tile = (n_oc + NS - 1) // NS

        def copy_out(i, _):
            k = s + i * NS

            @pl.when(k < n_oc)
            def _():
                pltpu.sync_copy(acc_sh.at[pl.ds(k * 1024, 1024)], zbuf)
                pltpu.sync_copy(
                    zbuf, out_hbm.at[pl.ds(c * n_nodes + k * 1024, 1024)]
                )
            return 0
        lax.fori_loop(0, oc_per_tile, copy_out, 0)
        rem = n_nodes - n_oc * 1024
        if rem:
            @pl.when(s == NS - 1)
            def _():
                pltpu.sync_copy(acc_sh.at[pl.ds(n_oc * 1024, rem)], zbuf.at[pl.ds(0, rem)])
                pltpu.sync_copy(
                    zbuf.at[pl.ds(0, rem)],
                    out_hbm.at[pl.ds(c * n_nodes + n_oc * 1024, rem)],
                )

    return deg_kernel


def _scatter_kernel(n_nodes, n_edges, d):
    edges_per_tile = n_edges // (NC * NS)
    n_full = edges_per_tile // CHUNK
    tail = edges_per_tile - n_full * CHUNK
    n_row_chunks = n_nodes // CHUNK
    row_tail = n_nodes - n_row_chunks * CHUNK
    chunks_per_tile = (n_row_chunks + NS - 1) // NS

    @functools.partial(
        pl.kernel,
        out_type=jax.ShapeDtypeStruct((NC * n_nodes, d), jnp.float32),
        mesh=_sc_mesh(),
        scratch_types=[
            pltpu.VMEM((CHUNK,), jnp.int32),
            pltpu.VMEM((CHUNK,), jnp.int32),
            pltpu.VMEM((CHUNK, d), jnp.float32),
            pltpu.VMEM((16,), jnp.int32),
            pltpu.VMEM((16,), jnp.int32),
            pltpu.VMEM_SHARED((n_nodes, d), jnp.float32),
            pltpu.SemaphoreType.DMA,
        ],
    )
    def scat_kernel(src_hbm, dst_hbm, g_hbm, out_hbm,
                    sidx, didx, rows, sidx_t, didx_t, acc_sh, sem):
        c = lax.axis_index("c")
        s = lax.axis_index("s")
        tile = c * NS + s
        base = tile * edges_per_tile

        def fill_row(i, _):
            def fill_lane(j, _):
                rows[i, pl.ds(j * LANES, LANES)] = jnp.zeros((LANES,), jnp.float32)
                return 0
            lax.fori_loop(0, d // LANES, fill_lane, 0)
            return 0
        lax.fori_loop(0, CHUNK, fill_row, 0)

        def zero_rows(i, _):
            k = s + i * NS

            @pl.when(k < n_row_chunks)
            def _():
                pltpu.sync_copy(rows, acc_sh.at[pl.ds(k * CHUNK, CHUNK)])
            return 0
        lax.fori_loop(0, chunks_per_tile, zero_rows, 0)
        if row_tail:
            @pl.when(s == 0)
            def _():
                pltpu.sync_copy(
                    rows.at[pl.ds(0, row_tail)],
                    acc_sh.at[pl.ds(n_row_chunks * CHUNK, row_tail)],
                )

        plsc.subcore_barrier()

        def body(j, _):
            off = base + j * CHUNK
            pltpu.sync_copy(src_hbm.at[pl.ds(off, CHUNK)], sidx)
            pltpu.sync_copy(dst_hbm.at[pl.ds(off, CHUNK)], didx)
            pltpu.async_copy(g_hbm.at[sidx], rows, sem).wait()
            pltpu.sync_copy(rows, acc_sh.at[didx], add=True)
            return 0
        lax.fori_loop(0, n_full, body, 0)

        if tail:
            off = base + n_full * CHUNK
            pltpu.sync_copy(src_hbm.at[pl.ds(off, tail)], sidx_t)
            pltpu.sync_copy(dst_hbm.at[pl.ds(off, tail)], didx_t)
            pltpu.async_copy(
                g_hbm.at[sidx_t], rows.at[pl.ds(0, tail)], sem
            ).wait()
            pltpu.sync_copy(rows.at[pl.ds(0, tail)], acc_sh.at[didx_t], add=True)

        plsc.subcore_barrier()

        out_base = c * n_nodes

        def copy_out(i, _):
            k = s + i * NS

            @pl.when(k < n_row_chunks)
            def _():
                pltpu.sync_copy(
                    acc_sh.at[pl.ds(k * CHUNK, CHUNK)],
                    out_hbm.at[pl.ds(out_base + k * CHUNK, CHUNK)],
                )
            return 0
        lax.fori_loop(0, chunks_per_tile, copy_out, 0)
        if row_tail:
            @pl.when(s == 0)
            def _():
                pltpu.sync_copy(
                    acc_sh.at[pl.ds(n_row_chunks * CHUNK, row_tail)],
                    out_hbm.at[pl.ds(out_base + n_row_chunks * CHUNK, row_tail)],
                )

    return scat_kernel


def _gW_body(x_ref, w_ref, degp_ref, g_ref):
    n = x_ref.shape[0]
    h = jnp.dot(x_ref[...], w_ref[...], preferred_element_type=jnp.float32)
    deg = degp_ref[0:n] + degp_ref[n:2 * n] + 1.0
    dis = lax.rsqrt(deg)
    g_ref[...] = h * dis[:, None]


def _epilogue_body(s_ref, g_ref, degp_ref, b_ref, gamma_ref, beta_ref, y_ref):
    n = g_ref.shape[0]
    deg = degp_ref[0:n] + degp_ref[n:2 * n] + 1.0
    dis = lax.rsqrt(deg)
    total = s_ref[0:n, :] + s_ref[n:2 * n, :] + g_ref[...]
    out = total * dis[:, None] + b_ref[...][None, :]
    mean = jnp.mean(out, axis=0)
    var = jnp.mean((out - mean[None, :]) ** 2, axis=0)
    y = gamma_ref[...][None, :] * (out - mean[None, :]) * lax.rsqrt(
        var[None, :] + 1e-5
    ) + beta_ref[...][None, :]
    y_ref[...] = jnp.maximum(y, 0.0)


def kernel(x, edge_index, W, b, gamma, beta):
    n_nodes, d_in = x.shape
    d_out = W.shape[1]
    n_edges = edge_index.shape[1]
    src = edge_index[0]
    dst = edge_index[1]

    n_tiles = NC * NS
    dst_deg = dst
    n_edges_deg = n_edges
    if n_edges_deg % n_tiles:
        flat_pad = n_tiles - n_edges_deg % n_tiles
        dst_deg = jnp.concatenate(
            [dst_deg, jnp.full((flat_pad,), n_nodes, jnp.int32)]
        )
        n_edges_deg += flat_pad
    per_tile = n_edges_deg // n_tiles
    n_chunks_deg = -(-per_tile // CHUNK)
    pad_pt = n_chunks_deg * CHUNK - per_tile
    dummy = n_nodes + jnp.arange(pad_pt, dtype=jnp.int32) % CHUNK
    dst3 = jnp.concatenate(
        [dst_deg.reshape(n_tiles, per_tile),
         jnp.broadcast_to(dummy[None, :], (n_tiles, pad_pt))], axis=1
    ).reshape(n_tiles, n_chunks_deg, CHUNK)

    degp = _degree_kernel(n_nodes, n_chunks_deg)(dst3)

    g = pl.pallas_call(
        _gW_body,
        out_shape=jax.ShapeDtypeStruct((n_nodes, d_out), jnp.float32),
    )(x, W, degp)

    s_partial = _scatter_kernel(n_nodes, n_edges, d_out)(src, dst, g)

    y = pl.pallas_call(
        _epilogue_body,
        out_shape=jax.ShapeDtypeStruct((n_nodes, d_out), jnp.float32),
    )(s_partial, g, degp, b, gamma, beta)
    return y


# speedup vs baseline: 2.8702x; 1.4928x over previous
import functools

import jax
import jax.numpy as jnp
from jax import lax
from jax.experimental import pallas as pl
from jax.experimental.pallas import tpu as pltpu
from jax.experimental.pallas import tpu_sc as plsc

NC = 2
NS = 16
LANES = 16

CHUNK = 128


def _sc_mesh():
    return plsc.VectorSubcoreMesh(
        core_axis_name="c", subcore_axis_name="s", num_cores=NC, num_subcores=NS
    )


def _degree_kernel(n_nodes, n_chunks):
    acc_len = n_nodes + CHUNK

    @functools.partial(
        pl.kernel,
        out_type=jax.ShapeDtypeStruct((NC * n_nodes,), jnp.float32),
        mesh=_sc_mesh(),
        scratch_types=[
            pltpu.VMEM((n_chunks, CHUNK), jnp.int32),
            pltpu.VMEM((CHUNK,), jnp.float32),
            pltpu.VMEM((1024,), jnp.float32),
            pltpu.VMEM_SHARED((acc_len,), jnp.float32),
            pltpu.SemaphoreType.DMA,
        ],
    )
    def deg_kernel(dst3_hbm, out_hbm, didx, ones_v, zbuf, acc_sh, sem):
        c = lax.axis_index("c")
        s = lax.axis_index("s")
        tile = c * NS + s

        pltpu.sync_copy(dst3_hbm.at[tile], didx)

        def fill_ones(i, _):
            ones_v[pl.ds(i * LANES, LANES)] = jnp.ones((LANES,), jnp.float32)
            return 0
        lax.fori_loop(0, CHUNK // LANES, fill_ones, 0)

        def fill_zero(i, _):
            zbuf[pl.ds(i * LANES, LANES)] = jnp.zeros((LANES,), jnp.float32)
            return 0
        lax.fori_loop(0, 1024 // LANES, fill_zero, 0)

        @pl.when(s == 0)
        def _():
            n_z = n_nodes // 1024

            def zero_acc(i, _):
                pltpu.sync_copy(zbuf, acc_sh.at[pl.ds(i * 1024, 1024)])
                return 0
            lax.fori_loop(0, n_z, zero_acc, 0)
            rem = n_nodes - n_z * 1024
            if rem:
                pltpu.sync_copy(
                    zbuf.at[pl.ds(0, rem)], acc_sh.at[pl.ds(n_z * 1024, rem)]
                )

        plsc.subcore_barrier()

        def fire(j, _):
            pltpu.async_copy(ones_v, acc_sh.at[didx.at[j]], sem, add=True)
            return 0
        lax.fori_loop(0, n_chunks, fire, 0)

        def drain(j, _):
            pltpu.make_async_copy(ones_v, acc_sh.at[didx.at[0]], sem).wait()
            return 0
        lax.fori_loop(0, n_chunks, drain, 0)

        plsc.subcore_barrier()

        n_oc = n_nodes // 1024
        oc_per_tile = (n_oc + NS - 1) // NS

        def copy_out(i, _):
            k = s + i * NS

            @pl.when(k < n_oc)
            def _():
                pltpu.sync_copy(acc_sh.at[pl.ds(k * 1024, 1024)], zbuf)
                pltpu.sync_copy(
                    zbuf, out_hbm.at[pl.ds(c * n_nodes + k * 1024, 1024)]
                )
            return 0
        lax.fori_loop(0, oc_per_tile, copy_out, 0)
        rem = n_nodes - n_oc * 1024
        if rem:
            @pl.when(s == NS - 1)
            def _():
                pltpu.sync_copy(acc_sh.at[pl.ds(n_oc * 1024, rem)], zbuf.at[pl.ds(0, rem)])
                pltpu.sync_copy(
                    zbuf.at[pl.ds(0, rem)],
                    out_hbm.at[pl.ds(c * n_nodes + n_oc * 1024, rem)],
                )

    return deg_kernel


def _scatter_kernel(n_nodes, n_edges, d):
    edges_per_tile = n_edges // (NC * NS)
    n_full = edges_per_tile // CHUNK
    tail = edges_per_tile - n_full * CHUNK
    n_row_chunks = n_nodes // CHUNK
    row_tail = n_nodes - n_row_chunks * CHUNK
    chunks_per_tile = (n_row_chunks + NS - 1) // NS

    @functools.partial(
        pl.kernel,
        out_type=jax.ShapeDtypeStruct((NC * n_nodes, d), jnp.float32),
        mesh=_sc_mesh(),
        scratch_types=[
            pltpu.VMEM((CHUNK,), jnp.int32),
            pltpu.VMEM((CHUNK,), jnp.int32),
            pltpu.VMEM((CHUNK,), jnp.int32),
            pltpu.VMEM((CHUNK,), jnp.int32),
            pltpu.VMEM((CHUNK, d), jnp.float32),
            pltpu.VMEM((CHUNK, d), jnp.float32),
            pltpu.VMEM((16,), jnp.int32),
            pltpu.VMEM((16,), jnp.int32),
            pltpu.VMEM_SHARED((n_nodes, d), jnp.float32),
            pltpu.SemaphoreType.DMA,
        ],
    )
    def scat_kernel(src_hbm, dst_hbm, g_hbm, out_hbm,
                    sidx, didx, sidx2, didx2, rows, rows2,
                    sidx_t, didx_t, acc_sh, sem):
        c = lax.axis_index("c")
        s = lax.axis_index("s")
        tile = c * NS + s
        base = tile * edges_per_tile

        def fill_row(i, _):
            def fill_lane(j, _):
                rows[i, pl.ds(j * LANES, LANES)] = jnp.zeros((LANES,), jnp.float32)
                return 0
            lax.fori_loop(0, d // LANES, fill_lane, 0)
            return 0
        lax.fori_loop(0, CHUNK, fill_row, 0)

        def zero_rows(i, _):
            k = s + i * NS

            @pl.when(k < n_row_chunks)
            def _():
                pltpu.sync_copy(rows, acc_sh.at[pl.ds(k * CHUNK, CHUNK)])
            return 0
        lax.fori_loop(0, chunks_per_tile, zero_rows, 0)
        if row_tail:
            @pl.when(s == 0)
            def _():
                pltpu.sync_copy(
                    rows.at[pl.ds(0, row_tail)],
                    acc_sh.at[pl.ds(n_row_chunks * CHUNK, row_tail)],
                )

        plsc.subcore_barrier()

        def wait_gather(buf):
            pltpu.make_async_copy(g_hbm.at[pl.ds(0, CHUNK)], buf, sem).wait()

        assert n_full % 2 == 0
        pltpu.sync_copy(src_hbm.at[pl.ds(base, CHUNK)], sidx)
        pltpu.async_copy(g_hbm.at[sidx], rows, sem)

        def body(i, _):
            o0 = base + (2 * i) * CHUNK
            o1 = base + (2 * i + 1) * CHUNK
            pltpu.sync_copy(src_hbm.at[pl.ds(o1, CHUNK)], sidx2)
            pltpu.async_copy(g_hbm.at[sidx2], rows2, sem)
            pltpu.sync_copy(dst_hbm.at[pl.ds(o0, CHUNK)], didx)
            wait_gather(rows)
            pltpu.sync_copy(rows, acc_sh.at[didx], add=True)
            @pl.when(2 * i + 2 < n_full)
            def _():
                pltpu.sync_copy(
                    src_hbm.at[pl.ds(o0 + 2 * CHUNK, CHUNK)], sidx
                )
                pltpu.async_copy(g_hbm.at[sidx], rows, sem)
            pltpu.sync_copy(dst_hbm.at[pl.ds(o1, CHUNK)], didx2)
            wait_gather(rows2)
            pltpu.sync_copy(rows2, acc_sh.at[didx2], add=True)
            return 0
        lax.fori_loop(0, n_full // 2, body, 0)

        if tail:
            off = base + n_full * CHUNK
            pltpu.sync_copy(src_hbm.at[pl.ds(off, tail)], sidx_t)
            pltpu.sync_copy(dst_hbm.at[pl.ds(off, tail)], didx_t)
            pltpu.async_copy(
                g_hbm.at[sidx_t], rows.at[pl.ds(0, tail)], sem
            ).wait()
            pltpu.sync_copy(rows.at[pl.ds(0, tail)], acc_sh.at[didx_t], add=True)

        plsc.subcore_barrier()

        out_base = c * n_nodes

        def copy_out(i, _):
            k = s + i * NS

            @pl.when(k < n_row_chunks)
            def _():
                pltpu.sync_copy(
                    acc_sh.at[pl.ds(k * CHUNK, CHUNK)],
                    out_hbm.at[pl.ds(out_base + k * CHUNK, CHUNK)],
                )
            return 0
        lax.fori_loop(0, chunks_per_tile, copy_out, 0)
        if row_tail:
            @pl.when(s == 0)
            def _():
                pltpu.sync_copy(
                    acc_sh.at[pl.ds(n_row_chunks * CHUNK, row_tail)],
                    out_hbm.at[pl.ds(out_base + n_row_chunks * CHUNK, row_tail)],
                )

    return scat_kernel


def _gW_body(x_ref, w_ref, degp_ref, g_ref):
    n = x_ref.shape[0]
    h = jnp.dot(x_ref[...], w_ref[...], preferred_element_type=jnp.float32)
    deg = degp_ref[0:n] + degp_ref[n:2 * n] + 1.0
    dis = lax.rsqrt(deg)
    g_ref[...] = h * dis[:, None]


def _epilogue_body(s_ref, g_ref, degp_ref, b_ref, gamma_ref, beta_ref, y_ref):
    n = g_ref.shape[0]
    deg = degp_ref[0:n] + degp_ref[n:2 * n] + 1.0
    dis = lax.rsqrt(deg)
    total = s_ref[0:n, :] + s_ref[n:2 * n, :] + g_ref[...]
    out = total * dis[:, None] + b_ref[...][None, :]
    mean = jnp.mean(out, axis=0)
    var = jnp.mean((out - mean[None, :]) ** 2, axis=0)
    y = gamma_ref[...][None, :] * (out - mean[None, :]) * lax.rsqrt(
        var[None, :] + 1e-5
    ) + beta_ref[...][None, :]
    y_ref[...] = jnp.maximum(y, 0.0)


def kernel(x, edge_index, W, b, gamma, beta):
    n_nodes, d_in = x.shape
    d_out = W.shape[1]
    n_edges = edge_index.shape[1]
    src = edge_index[0]
    dst = edge_index[1]

    n_tiles = NC * NS
    dst_deg = dst
    n_edges_deg = n_edges
    if n_edges_deg % n_tiles:
        flat_pad = n_tiles - n_edges_deg % n_tiles
        dst_deg = jnp.concatenate(
            [dst_deg, jnp.full((flat_pad,), n_nodes, jnp.int32)]
        )
        n_edges_deg += flat_pad
    per_tile = n_edges_deg // n_tiles
    n_chunks_deg = -(-per_tile // CHUNK)
    pad_pt = n_chunks_deg * CHUNK - per_tile
    dummy = n_nodes + jnp.arange(pad_pt, dtype=jnp.int32) % CHUNK
    dst3 = jnp.concatenate(
        [dst_deg.reshape(n_tiles, per_tile),
         jnp.broadcast_to(dummy[None, :], (n_tiles, pad_pt))], axis=1
    ).reshape(n_tiles, n_chunks_deg, CHUNK)

    degp = _degree_kernel(n_nodes, n_chunks_deg)(dst3)

    g = pl.pallas_call(
        _gW_body,
        out_shape=jax.ShapeDtypeStruct((n_nodes, d_out), jnp.float32),
    )(x, W, degp)

    s_partial = _scatter_kernel(n_nodes, n_edges, d_out)(src, dst, g)

    y = pl.pallas_call(
        _epilogue_body,
        out_shape=jax.ShapeDtypeStruct((n_nodes, d_out), jnp.float32),
    )(s_partial, g, degp, b, gamma, beta)
    return y


# speedup vs baseline: 3.2158x; 1.1204x over previous
import functools

import jax
import jax.numpy as jnp
from jax import lax
from jax.experimental import pallas as pl
from jax.experimental.pallas import tpu as pltpu
from jax.experimental.pallas import tpu_sc as plsc

NC = 2
NS = 16
LANES = 16

CHUNK = 128


def _sc_mesh():
    return plsc.VectorSubcoreMesh(
        core_axis_name="c", subcore_axis_name="s", num_cores=NC, num_subcores=NS
    )


def _degree_kernel(n_nodes, n_chunks):
    acc_len = n_nodes + CHUNK

    @functools.partial(
        pl.kernel,
        out_type=jax.ShapeDtypeStruct((NC * n_nodes,), jnp.float32),
        mesh=_sc_mesh(),
        scratch_types=[
            pltpu.VMEM((n_chunks, CHUNK), jnp.int32),
            pltpu.VMEM((CHUNK,), jnp.float32),
            pltpu.VMEM((1024,), jnp.float32),
            pltpu.VMEM_SHARED((acc_len,), jnp.float32),
            pltpu.SemaphoreType.DMA,
        ],
    )
    def deg_kernel(dst3_hbm, out_hbm, didx, ones_v, zbuf, acc_sh, sem):
        c = lax.axis_index("c")
        s = lax.axis_index("s")
        tile = c * NS + s

        pltpu.sync_copy(dst3_hbm.at[tile], didx)

        def fill_ones(i, _):
            ones_v[pl.ds(i * LANES, LANES)] = jnp.ones((LANES,), jnp.float32)
            return 0
        lax.fori_loop(0, CHUNK // LANES, fill_ones, 0)

        def fill_zero(i, _):
            zbuf[pl.ds(i * LANES, LANES)] = jnp.zeros((LANES,), jnp.float32)
            return 0
        lax.fori_loop(0, 1024 // LANES, fill_zero, 0)

        @pl.when(s == 0)
        def _():
            n_z = n_nodes // 1024

            def zero_acc(i, _):
                pltpu.sync_copy(zbuf, acc_sh.at[pl.ds(i * 1024, 1024)])
                return 0
            lax.fori_loop(0, n_z, zero_acc, 0)
            rem = n_nodes - n_z * 1024
            if rem:
                pltpu.sync_copy(
                    zbuf.at[pl.ds(0, rem)], acc_sh.at[pl.ds(n_z * 1024, rem)]
                )

        plsc.subcore_barrier()

        def fire(j, _):
            pltpu.async_copy(ones_v, acc_sh.at[didx.at[j]], sem, add=True)
            return 0
        lax.fori_loop(0, n_chunks, fire, 0)

        def drain(j, _):
            pltpu.make_async_copy(ones_v, acc_sh.at[didx.at[0]], sem).wait()
            return 0
        lax.fori_loop(0, n_chunks, drain, 0)

        plsc.subcore_barrier()

        n_oc = n_nodes // 1024
        oc_per_tile = (n_oc + NS - 1) // NS

        def copy_out(i, _):
            k = s + i * NS

            @pl.when(k < n_oc)
            def _():
                pltpu.sync_copy(acc_sh.at[pl.ds(k * 1024, 1024)], zbuf)
                pltpu.sync_copy(
                    zbuf, out_hbm.at[pl.ds(c * n_nodes + k * 1024, 1024)]
                )
            return 0
        lax.fori_loop(0, oc_per_tile, copy_out, 0)
        rem = n_nodes - n_oc * 1024
        if rem:
            @pl.when(s == NS - 1)
            def _():
                pltpu.sync_copy(acc_sh.at[pl.ds(n_oc * 1024, rem)], zbuf.at[pl.ds(0, rem)])
                pltpu.sync_copy(
                    zbuf.at[pl.ds(0, rem)],
                    out_hbm.at[pl.ds(c * n_nodes + n_oc * 1024, rem)],
                )

    return deg_kernel


def _scatter_kernel(n_nodes, n_edges, d):
    edges_per_tile = n_edges // (NC * NS)
    n_full = edges_per_tile // CHUNK
    tail = edges_per_tile - n_full * CHUNK
    n_row_chunks = n_nodes // CHUNK
    row_tail = n_nodes - n_row_chunks * CHUNK
    chunks_per_tile = (n_row_chunks + NS - 1) // NS

    @functools.partial(
        pl.kernel,
        out_type=jax.ShapeDtypeStruct((NC * n_nodes, d), jnp.float32),
        mesh=_sc_mesh(),
        scratch_types=[
            pltpu.VMEM((CHUNK,), jnp.int32),
            pltpu.VMEM((CHUNK,), jnp.int32),
            pltpu.VMEM((CHUNK,), jnp.int32),
            pltpu.VMEM((CHUNK,), jnp.int32),
            pltpu.VMEM((CHUNK,), jnp.int32),
            pltpu.VMEM((CHUNK,), jnp.int32),
            pltpu.VMEM((CHUNK, d), jnp.float32),
            pltpu.VMEM((CHUNK, d), jnp.float32),
            pltpu.VMEM((CHUNK, d), jnp.float32),
            pltpu.VMEM((16,), jnp.int32),
            pltpu.VMEM((16,), jnp.int32),
            pltpu.VMEM_SHARED((n_nodes, d), jnp.float32),
            pltpu.SemaphoreType.DMA,
            pltpu.SemaphoreType.DMA,
        ],
    )
    def scat_kernel(src_hbm, dst_hbm, g_hbm, out_hbm,
                    sidx_a, didx_a, sidx_b, didx_b, sidx_c, didx_c,
                    rows_a, rows_b, rows_c,
                    sidx_t, didx_t, acc_sh, sem, ssem):
        c = lax.axis_index("c")
        s = lax.axis_index("s")
        tile = c * NS + s
        base = tile * edges_per_tile

        def fill_row(i, _):
            def fill_lane(j, _):
                rows_a[i, pl.ds(j * LANES, LANES)] = jnp.zeros((LANES,), jnp.float32)
                return 0
            lax.fori_loop(0, d // LANES, fill_lane, 0)
            return 0
        lax.fori_loop(0, CHUNK, fill_row, 0)

        def zero_rows(i, _):
            k = s + i * NS

            @pl.when(k < n_row_chunks)
            def _():
                pltpu.sync_copy(rows_a, acc_sh.at[pl.ds(k * CHUNK, CHUNK)])
            return 0
        lax.fori_loop(0, chunks_per_tile, zero_rows, 0)
        if row_tail:
            @pl.when(s == 0)
            def _():
                pltpu.sync_copy(
                    rows_a.at[pl.ds(0, row_tail)],
                    acc_sh.at[pl.ds(n_row_chunks * CHUNK, row_tail)],
                )

        plsc.subcore_barrier()

        def wait_gather(buf):
            pltpu.make_async_copy(g_hbm.at[pl.ds(0, CHUNK)], buf, sem).wait()

        def wait_scatter(buf, didx):
            pltpu.make_async_copy(buf, acc_sh.at[didx], ssem).wait()

        assert n_full % 3 == 0
        pltpu.sync_copy(src_hbm.at[pl.ds(base, CHUNK)], sidx_a)
        pltpu.async_copy(g_hbm.at[sidx_a], rows_a, sem)
        pltpu.sync_copy(src_hbm.at[pl.ds(base + CHUNK, CHUNK)], sidx_b)
        pltpu.async_copy(g_hbm.at[sidx_b], rows_b, sem)

        def body(i, _):
            j0 = base + (3 * i) * CHUNK
            j1 = base + (3 * i + 1) * CHUNK
            j2 = base + (3 * i + 2) * CHUNK
            pltpu.sync_copy(dst_hbm.at[pl.ds(j0, CHUNK)], didx_a)
            wait_gather(rows_a)
            pltpu.async_copy(rows_a, acc_sh.at[didx_a], ssem, add=True)
            pltpu.sync_copy(dst_hbm.at[pl.ds(j1, CHUNK)], didx_b)
            wait_gather(rows_b)
            pltpu.async_copy(rows_b, acc_sh.at[didx_b], ssem, add=True)
            @pl.when(i > 0)
            def _():
                wait_scatter(rows_c, didx_c)
            pltpu.sync_copy(src_hbm.at[pl.ds(j2, CHUNK)], sidx_c)
            pltpu.async_copy(g_hbm.at[sidx_c], rows_c, sem)
            wait_scatter(rows_a, didx_a)

            @pl.when(3 * i + 3 < n_full)
            def _():
                pltpu.sync_copy(src_hbm.at[pl.ds(j2 + CHUNK, CHUNK)], sidx_a)
                pltpu.async_copy(g_hbm.at[sidx_a], rows_a, sem)
            pltpu.sync_copy(dst_hbm.at[pl.ds(j2, CHUNK)], didx_c)
            wait_gather(rows_c)
            pltpu.async_copy(rows_c, acc_sh.at[didx_c], ssem, add=True)
            wait_scatter(rows_b, didx_b)

            @pl.when(3 * i + 4 < n_full)
            def _():
                pltpu.sync_copy(
                    src_hbm.at[pl.ds(j2 + 2 * CHUNK, CHUNK)], sidx_b
                )
                pltpu.async_copy(g_hbm.at[sidx_b], rows_b, sem)
            return 0
        lax.fori_loop(0, n_full // 3, body, 0)
        wait_scatter(rows_c, didx_c)

        if tail:
            off = base + n_full * CHUNK
            pltpu.sync_copy(src_hbm.at[pl.ds(off, tail)], sidx_t)
            pltpu.sync_copy(dst_hbm.at[pl.ds(off, tail)], didx_t)
            pltpu.async_copy(
                g_hbm.at[sidx_t], rows_a.at[pl.ds(0, tail)], sem
            ).wait()
            pltpu.sync_copy(rows_a.at[pl.ds(0, tail)], acc_sh.at[didx_t],
                            add=True)

        plsc.subcore_barrier()

        out_base = c * n_nodes

        def copy_out(i, _):
            k = s + i * NS

            @pl.when(k < n_row_chunks)
            def _():
                pltpu.sync_copy(
                    acc_sh.at[pl.ds(k * CHUNK, CHUNK)],
                    out_hbm.at[pl.ds(out_base + k * CHUNK, CHUNK)],
                )
            return 0
        lax.fori_loop(0, chunks_per_tile, copy_out, 0)
        if row_tail:
            @pl.when(s == 0)
            def _():
                pltpu.sync_copy(
                    acc_sh.at[pl.ds(n_row_chunks * CHUNK, row_tail)],
                    out_hbm.at[pl.ds(out_base + n_row_chunks * CHUNK, row_tail)],
                )

    return scat_kernel


def _gW_body(x_ref, w_ref, degp_ref, g_ref):
    n = x_ref.shape[0]
    h = jnp.dot(x_ref[...], w_ref[...], preferred_element_type=jnp.float32)
    deg = degp_ref[0:n] + degp_ref[n:2 * n] + 1.0
    dis = lax.rsqrt(deg)
    g_ref[...] = h * dis[:, None]


def _epilogue_body(s_ref, g_ref, degp_ref, b_ref, gamma_ref, beta_ref, y_ref):
    n = g_ref.shape[0]
    deg = degp_ref[0:n] + degp_ref[n:2 * n] + 1.0
    dis = lax.rsqrt(deg)
    total = s_ref[0:n, :] + s_ref[n:2 * n, :] + g_ref[...]
    out = total * dis[:, None] + b_ref[...][None, :]
    mean = jnp.mean(out, axis=0)
    var = jnp.mean((out - mean[None, :]) ** 2, axis=0)
    y = gamma_ref[...][None, :] * (out - mean[None, :]) * lax.rsqrt(
        var[None, :] + 1e-5
    ) + beta_ref[...][None, :]
    y_ref[...] = jnp.maximum(y, 0.0)


def kernel(x, edge_index, W, b, gamma, beta):
    n_nodes, d_in = x.shape
    d_out = W.shape[1]
    n_edges = edge_index.shape[1]
    src = edge_index[0]
    dst = edge_index[1]

    n_tiles = NC * NS
    dst_deg = dst
    n_edges_deg = n_edges
    if n_edges_deg % n_tiles:
        flat_pad = n_tiles - n_edges_deg % n_tiles
        dst_deg = jnp.concatenate(
            [dst_deg, jnp.full((flat_pad,), n_nodes, jnp.int32)]
        )
        n_edges_deg += flat_pad
    per_tile = n_edges_deg // n_tiles
    n_chunks_deg = -(-per_tile // CHUNK)
    pad_pt = n_chunks_deg * CHUNK - per_tile
    dummy = n_nodes + jnp.arange(pad_pt, dtype=jnp.int32) % CHUNK
    dst3 = jnp.concatenate(
        [dst_deg.reshape(n_tiles, per_tile),
         jnp.broadcast_to(dummy[None, :], (n_tiles, pad_pt))], axis=1
    ).reshape(n_tiles, n_chunks_deg, CHUNK)

    degp = _degree_kernel(n_nodes, n_chunks_deg)(dst3)

    g = pl.pallas_call(
        _gW_body,
        out_shape=jax.ShapeDtypeStruct((n_nodes, d_out), jnp.float32),
    )(x, W, degp)

    s_partial = _scatter_kernel(n_nodes, n_edges, d_out)(src, dst, g)

    y = pl.pallas_call(
        _epilogue_body,
        out_shape=jax.ShapeDtypeStruct((n_nodes, d_out), jnp.float32),
    )(s_partial, g, degp, b, gamma, beta)
    return y
